# baseline (speedup 1.0000x reference)
import jax
import jax.numpy as jnp
from jax import lax
from jax.experimental import pallas as pl
from jax.experimental.pallas import tpu as pltpu
from jax.experimental.pallas import tpu_sc as plsc

B = 16384
D = 64
NC, NS = 2, 16
NW = NC * NS
BPW = B // NW
CH = 16
NCH = BPW // CH
L = 16
G = 8


def _body(ux_hbm, ix_hbm, uid_hbm, iid_hbm, out_hbm,
          uxv, ixv, bu0, bi0, bu1, bi1, out_v, sem):
    wid = lax.axis_index("s") * NC + lax.axis_index("c")
    base = wid * BPW

    pltpu.sync_copy(ux_hbm.at[pl.ds(base, BPW)], uxv)
    pltpu.sync_copy(ix_hbm.at[pl.ds(base, BPW)], ixv)

    lanes = lax.iota(jnp.int32, L)
    perms = [lanes ^ sh for sh in (8, 4, 2, 1)]
    u3 = uid_hbm.reshape(100000 // G, G, D)

    i3 = iid_hbm.at[pl.ds(0, 100000)].reshape(100000 // G, G, D)

    def fire(c, bu, bi):
        cb = c * CH
        for h in range(CH // L):
            tq = pl.ds(cb + h * L, L)
            tu16 = lax.shift_right_logical(uxv[tq], 3)
            ti16 = lax.shift_right_logical(ixv[tq], 3)
            for jj in range(L):
                jc = h * L + jj
                tu = tu16[jj]
                ti = ti16[jj]
                pltpu.async_copy(u3.at[tu], bu.at[jc], sem)
                pltpu.async_copy(i3.at[ti], bi.at[jc], sem)

    def drain(bu, bi):
        pltpu.make_async_copy(u3.at[pl.ds(0, CH)], bu, sem).wait()
        pltpu.make_async_copy(u3.at[pl.ds(0, CH)], bi, sem).wait()

    def compute(c, bu, bi):
        cb = c * CH
        for g in range(CH // L):
            r0 = g * L
            sq = pl.ds(cb + r0, L)
            su16 = uxv[sq] & 7
            si16 = ixv[sq] & 7
            out16 = jnp.zeros((L,), jnp.float32)
            for jj in range(L):
                jc = r0 + jj
                su = su16[jj]
                si = si16[jj]
                acc = None
                for k in range(D // L):
                    u = bu[jc, su, pl.ds(k * L, L)]
                    v = bi[jc, si, pl.ds(k * L, L)]
                    p = u * v
                    acc = p if acc is None else acc + p
                for p_ in perms:
                    acc = acc + jnp.take_along_axis(
                        acc, p_, axis=0, mode="promise_in_bounds")
                out16 = jnp.where(lanes == jj, acc, out16)
            out_v[pl.ds(cb + r0, L)] = out16

    fire(0, bu0, bi0)

    def pair(i, carry):
        c0 = i * 2
        c1 = c0 + 1
        fire(c1, bu1, bi1)
        drain(bu0, bi0)
        compute(c0, bu0, bi0)
        c2 = jnp.minimum(c0 + 2, NCH - 1)
        fire(c2, bu0, bi0)
        drain(bu1, bi1)
        compute(c1, bu1, bi1)
        return carry

    lax.fori_loop(0, NCH // 2, pair, 0)
    drain(bu0, bi0)
    pltpu.sync_copy(out_v, out_hbm.at[pl.ds(base, BPW)])


def kernel(x, uid_table, iid_table):
    ux = x[:, 0]
    ix = x[:, 1]
    mesh = plsc.VectorSubcoreMesh(
        core_axis_name="c", subcore_axis_name="s",
        num_cores=NC, num_subcores=NS)
    run = pl.kernel(
        _body,
        out_type=jax.ShapeDtypeStruct((B,), jnp.float32),
        mesh=mesh,
        compiler_params=pltpu.CompilerParams(
            needs_layout_passes=False, use_tc_tiling_on_sc=True),
        scratch_types=[
            pltpu.VMEM((BPW,), jnp.int32),
            pltpu.VMEM((BPW,), jnp.int32),
            pltpu.VMEM((CH, G, D), jnp.float32),
            pltpu.VMEM((CH, G, D), jnp.float32),
            pltpu.VMEM((CH, G, D), jnp.float32),
            pltpu.VMEM((CH, G, D), jnp.float32),
            pltpu.VMEM((BPW,), jnp.float32),
            pltpu.SemaphoreType.DMA,
        ],
    )
    return run(ux, ix, uid_table, iid_table)
